# baseline (speedup 1.0000x reference)
import functools

import jax
import jax.numpy as jnp
from jax import lax
from jax.experimental import pallas as pl
from jax.experimental.pallas import tpu as pltpu
from jax.experimental.pallas import tpu_sc as plsc

S = 7
B = 2
C = 20
CH = B * 5 + C
BS = 8192
N_CELLS = BS * S * S
NC = 2
NS = 16
NW = NC * NS
L = 16
RPW = N_CELLS // NW
CHUNK = 784
NCHUNK = RPW // CHUNK
GROUPS = CHUNK // L
CW = CHUNK * CH
LN2 = 0.6931471805599453
LAMBDA_COORD = 5.0
LAMBDA_NOOBJ = 0.5


def _fsqrt(x):
    b = plsc.bitcast(x, jnp.int32)
    y = plsc.bitcast((b >> 1) + 0x1FBD1DF5, jnp.float32)
    y = 0.5 * (y + x / y)
    y = 0.5 * (y + x / y)
    y = 0.5 * (y + x / y)
    return y


def _flog(x):
    b = plsc.bitcast(x, jnp.int32)
    e = ((b >> 23) - 127).astype(jnp.float32)
    m = plsc.bitcast((b & 0x007FFFFF) | 0x3F800000, jnp.float32)
    t = (m - 1.0) / (m + 1.0)
    t2 = t * t
    p = 2.0 * t * (1.0 + t2 * (1.0 / 3.0 + t2 * (0.2 + t2 * (1.0 / 7.0 + t2 * (1.0 / 9.0)))))
    return e * LN2 + p


def _iou(bx, by, bw, bh, cx, cy, cw, ch):
    b1x1 = bx - bw / 2
    b1y1 = by - bh / 2
    b1x2 = bx + bw / 2
    b1y2 = by + bh / 2
    b2x1 = cx - cw / 2
    b2y1 = cy - ch / 2
    b2x2 = cx + cw / 2
    b2y2 = cy + ch / 2
    ix1 = jnp.maximum(b1x1, b2x1)
    iy1 = jnp.maximum(b1y1, b2y1)
    ix2 = jnp.minimum(b1x2, b2x2)
    iy2 = jnp.minimum(b1y2, b2y2)
    inter = jnp.maximum(ix2 - ix1, 0.0) * jnp.maximum(iy2 - iy1, 0.0)
    a1 = jnp.abs((b1x2 - b1x1) * (b1y2 - b1y1))
    a2 = jnp.abs((b2x2 - b2x1) * (b2y2 - b2y1))
    return inter / (a1 + a2 - inter + 1e-6)


def _group(pbuf, gbuf, i30, ibase, accs):
    idx0 = i30 + ibase

    def P(c):
        return plsc.load_gather(pbuf, [idx0 + c])

    def G(c):
        return plsc.load_gather(gbuf, [idx0 + c])

    p = [P(c) for c in range(CH)]
    g = [G(c) for c in range(9)]
    gcls = [G(c) for c in range(10, CH)]

    iou0 = _iou(p[0], p[1], p[2], p[3], g[0], g[1], g[2], g[3])
    iou1 = _iou(p[5], p[6], p[7], p[8], g[5], g[6], g[7], g[8])
    pick1 = iou1 > iou0
    src0 = g[4] > 0.0
    o0 = jnp.where(jnp.logical_and(jnp.logical_not(pick1), src0), 1.0, 0.0)
    o1 = jnp.where(jnp.logical_and(pick1, src0), 1.0, 0.0)

    def sq(v):
        return v * v

    xy = o0 * (sq(p[0] - g[0]) + sq(p[1] - g[1])) + \
         o1 * (sq(p[5] - g[5]) + sq(p[6] - g[6]))

    sp2 = _fsqrt(jnp.maximum(p[2], 1e-6))
    sp3 = _fsqrt(jnp.maximum(p[3], 1e-6))
    sp7 = _fsqrt(jnp.maximum(p[7], 1e-6))
    sp8 = _fsqrt(jnp.maximum(p[8], 1e-6))
    sg2 = _fsqrt(jnp.maximum(g[2], 1e-6))
    sg3 = _fsqrt(jnp.maximum(g[3], 1e-6))
    sg7 = _fsqrt(jnp.maximum(g[7], 1e-6))
    sg8 = _fsqrt(jnp.maximum(g[8], 1e-6))
    wh = o0 * (sq(sp2 - sg2) + sq(sp3 - sg3)) + \
         o1 * (sq(sp7 - sg7) + sq(sp8 - sg8))

    oc = o0 * sq(p[4] - g[4]) + o1 * sq(p[9] - g[5])
    pc2 = p[4] * p[4] + p[9] * p[9]
    pc2o = o0 * p[4] * p[4] + o1 * p[9] * p[9]
    cellf = jnp.where((g[4] + g[5]) > 0.0, 1.0, 0.0)

    m = p[10]
    for c in range(11, CH):
        m = jnp.maximum(m, p[c])
    ssum = lax.exp(p[10] - m)
    for c in range(11, CH):
        ssum = ssum + lax.exp(p[c] - m)
    lse = _flog(ssum) + m

    bg = gcls[0]
    bi = jnp.zeros((L,), jnp.int32)
    for c in range(1, C):
        cond = gcls[c] > bg
        bg = jnp.where(cond, gcls[c], bg)
        bi = jnp.where(cond, c, bi)
    ptgt = plsc.load_gather(pbuf, [idx0 + 10 + bi])
    nll = cellf * (lse - ptgt)

    cnt, a_xy, a_wh, a_oc, a_pc2, a_pc2o, a_cell, a_nll = accs
    return (cnt + (o0 + o1), a_xy + xy, a_wh + wh, a_oc + oc,
            a_pc2 + pc2, a_pc2o + pc2o, a_cell + cellf, a_nll + nll)


def _sc_body(pred_hbm, gt_hbm, out_hbm, pbuf, gbuf, obuf, sem_p, sem_g):
    wid = lax.axis_index("s") * NC + lax.axis_index("c")
    base = wid * (RPW * CH)
    i30 = lax.iota(jnp.int32, L) * CH

    def chunk_body(ci, accs):
        off = base + ci * CW
        cp = pltpu.async_copy(pred_hbm.at[pl.ds(off, CW)], pbuf, sem_p)
        cg = pltpu.async_copy(gt_hbm.at[pl.ds(off, CW)], gbuf, sem_g)
        cp.wait()
        cg.wait()

        def gb(gi, a):
            return _group(pbuf, gbuf, i30, gi * (CH * L), a)

        return lax.fori_loop(0, GROUPS, gb, accs)

    z = jnp.zeros((L,), jnp.float32)
    accs = lax.fori_loop(0, NCHUNK, chunk_body, (z,) * 8)
    for k in range(8):
        obuf[pl.ds(k * L, L)] = accs[k]
    pltpu.sync_copy(obuf, out_hbm.at[wid])


_sc_loss = functools.partial(
    pl.kernel,
    out_type=jax.ShapeDtypeStruct((NW, 8 * L), jnp.float32),
    mesh=plsc.VectorSubcoreMesh(
        core_axis_name="c", subcore_axis_name="s",
        num_cores=NC, num_subcores=NS),
    compiler_params=pltpu.CompilerParams(
        use_tc_tiling_on_sc=False, needs_layout_passes=False),
    scratch_types=[
        pltpu.VMEM((CW,), jnp.float32),
        pltpu.VMEM((CW,), jnp.float32),
        pltpu.VMEM((8 * L,), jnp.float32),
        pltpu.SemaphoreType.DMA,
        pltpu.SemaphoreType.DMA,
    ],
)(_sc_body)


def _fin_body(x_ref, o_ref):
    x = x_ref[...]
    s = [jnp.sum(x[:, k * L:(k + 1) * L]) for k in range(8)]
    cnt_obj, s_xy, s_wh, s_oc, s_pc2, s_pc2o, s_cell, s_nll = s
    cnt_noobj = float(N_CELLS * B) - cnt_obj
    xy_loss = s_xy / (2.0 * cnt_obj)
    wh_loss = s_wh / (2.0 * cnt_obj)
    loc_loss = LAMBDA_COORD * (xy_loss + wh_loss)
    conf_loss = s_oc / cnt_obj + LAMBDA_NOOBJ * (s_pc2 - s_pc2o) / cnt_noobj
    class_loss = s_nll / s_cell
    o_ref[0, 0] = (loc_loss + conf_loss + class_loss) / float(BS)


_finish = pl.pallas_call(
    _fin_body,
    out_shape=jax.ShapeDtypeStruct((1, 1), jnp.float32),
    out_specs=pl.BlockSpec(memory_space=pltpu.SMEM),
)


@jax.jit
def _run(pred, gt):
    p = pred.reshape(N_CELLS * CH)
    g = gt.reshape(N_CELLS * CH)
    partials = _sc_loss(p, g)
    return _finish(partials)[0, 0]


def kernel(pred, gt):
    return _run(pred, gt)
